# Initial kernel scaffold; baseline (speedup 1.0000x reference)
import jax
import jax.numpy as jnp
from jax.experimental import pallas as pl


def kernel(feat, edge_index):
    raise NotImplementedError("write your pallas kernel here")


# speedup vs baseline: 6.7290x; 6.7290x over previous
import functools

import jax
import jax.numpy as jnp
from jax import lax
from jax.experimental import pallas as pl
from jax.experimental.pallas import tpu as pltpu
from jax.experimental.pallas import tpu_sc as plsc

N_NODES = 10000
N_EDGES = 320000
D_FEAT = 128

NC = 2
NS = 16
NW = NC * NS

EPT = N_EDGES // NW
CHUNK = 128
NFULL = EPT // CHUNK
REM = EPT - NFULL * CHUNK
ROWS_PER_TILE = 624
ROWS_TAIL = N_NODES - NS * ROWS_PER_TILE


@functools.partial(
    pl.kernel,
    out_type=jax.ShapeDtypeStruct((NC * N_NODES, D_FEAT), jnp.float32),
    mesh=plsc.VectorSubcoreMesh(core_axis_name="c", subcore_axis_name="s"),
    scratch_types=[
        pltpu.VMEM_SHARED((N_NODES, D_FEAT), jnp.float32),
        pltpu.VMEM((CHUNK,), jnp.int32),
        pltpu.VMEM((CHUNK,), jnp.int32),
        pltpu.VMEM((CHUNK, D_FEAT), jnp.float32),
        pltpu.VMEM((REM,), jnp.int32),
        pltpu.VMEM((REM,), jnp.int32),
        pltpu.VMEM((REM, D_FEAT), jnp.float32),
        pltpu.SemaphoreType.DMA,
    ],
)
def _gin_scatter_sc(feat_hbm, src_hbm, dst_hbm, out_hbm,
                    accum, src_idx, dst_idx, rows,
                    src_tail, dst_tail, rows_tail, sem):
    c = lax.axis_index("c")
    s = lax.axis_index("s")
    wid = c * NS + s

    r0 = s * ROWS_PER_TILE
    pltpu.sync_copy(feat_hbm.at[pl.ds(r0, ROWS_PER_TILE)],
                    accum.at[pl.ds(r0, ROWS_PER_TILE)])

    @pl.when(s == NS - 1)
    def _init_tail():
        t0 = NS * ROWS_PER_TILE
        pltpu.sync_copy(feat_hbm.at[pl.ds(t0, ROWS_TAIL)],
                        accum.at[pl.ds(t0, ROWS_TAIL)])

    plsc.subcore_barrier()

    base = wid * EPT

    @pl.loop(0, NFULL)
    def _chunk_loop(k):
        off = base + k * CHUNK
        pltpu.sync_copy(src_hbm.at[pl.ds(off, CHUNK)], src_idx)
        pltpu.sync_copy(dst_hbm.at[pl.ds(off, CHUNK)], dst_idx)
        pltpu.async_copy(feat_hbm.at[src_idx], rows, sem).wait()
        pltpu.sync_copy(rows, accum.at[dst_idx], add=True)

    off = base + NFULL * CHUNK
    pltpu.sync_copy(src_hbm.at[pl.ds(off, REM)], src_tail)
    pltpu.sync_copy(dst_hbm.at[pl.ds(off, REM)], dst_tail)
    pltpu.async_copy(feat_hbm.at[src_tail], rows_tail, sem).wait()
    pltpu.sync_copy(rows_tail, accum.at[dst_tail], add=True)

    plsc.subcore_barrier()

    pltpu.sync_copy(accum.at[pl.ds(r0, ROWS_PER_TILE)],
                    out_hbm.at[pl.ds(c * N_NODES + r0, ROWS_PER_TILE)])

    @pl.when(s == NS - 1)
    def _out_tail():
        t0 = NS * ROWS_PER_TILE
        pltpu.sync_copy(accum.at[pl.ds(t0, ROWS_TAIL)],
                        out_hbm.at[pl.ds(c * N_NODES + t0, ROWS_TAIL)])


_COMBINE_BLOCK = 1000
_COMBINE_GRID = N_NODES // _COMBINE_BLOCK


def _combine_body(p0_ref, p1_ref, feat_ref, out_ref):
    out_ref[...] = p0_ref[...] + p1_ref[...] - feat_ref[...]


def _combine(partials, feat):
    return pl.pallas_call(
        _combine_body,
        grid=(_COMBINE_GRID,),
        in_specs=[
            pl.BlockSpec((_COMBINE_BLOCK, D_FEAT), lambda k: (k, 0)),
            pl.BlockSpec((_COMBINE_BLOCK, D_FEAT),
                         lambda k: (k + _COMBINE_GRID, 0)),
            pl.BlockSpec((_COMBINE_BLOCK, D_FEAT), lambda k: (k, 0)),
        ],
        out_specs=pl.BlockSpec((_COMBINE_BLOCK, D_FEAT), lambda k: (k, 0)),
        out_shape=jax.ShapeDtypeStruct((N_NODES, D_FEAT), jnp.float32),
    )(partials, partials, feat)


def kernel(feat, edge_index):
    src = edge_index[0]
    dst = edge_index[1]
    partials = _gin_scatter_sc(feat, src, dst)
    return _combine(partials, feat)


# speedup vs baseline: 13.5043x; 2.0069x over previous
import functools

import jax
import jax.numpy as jnp
from jax import lax
from jax.experimental import pallas as pl
from jax.experimental.pallas import tpu as pltpu
from jax.experimental.pallas import tpu_sc as plsc

N_NODES = 10000
N_EDGES = 320000
D_FEAT = 128

NC = 2
NS = 16
NW = NC * NS

EPT = N_EDGES // NW
CHUNK = 128
NFULL = EPT // CHUNK
TAIL = EPT - NFULL * CHUNK
NBUF = 3

ROWS_PER_TILE = 624
ROWS_TAIL = N_NODES - NS * ROWS_PER_TILE


@functools.partial(
    pl.kernel,
    out_type=jax.ShapeDtypeStruct((NC * N_NODES, D_FEAT), jnp.float32),
    mesh=plsc.VectorSubcoreMesh(core_axis_name="c", subcore_axis_name="s"),
    scratch_types=[
        pltpu.VMEM_SHARED((N_NODES, D_FEAT), jnp.float32),
    ]
    + [pltpu.VMEM((CHUNK, D_FEAT), jnp.float32) for _ in range(NBUF)]
    + [pltpu.VMEM((CHUNK,), jnp.int32) for _ in range(NBUF)]
    + [pltpu.VMEM((CHUNK,), jnp.int32) for _ in range(NBUF)]
    + [
        pltpu.VMEM((TAIL,), jnp.int32),
        pltpu.SemaphoreType.DMA((NBUF,)),
        pltpu.SemaphoreType.DMA((NBUF,)),
        pltpu.SemaphoreType.DMA((NBUF,)),
        pltpu.SemaphoreType.DMA((NBUF,)),
    ],
)
def _gin_scatter_sc(feat_hbm, src_hbm, dst_hbm, out_hbm,
                    accum, rb0, rb1, rb2, si0, si1, si2, di0, di1, di2,
                    didx_tail, sem_g, sem_s, sem_si, sem_di):
    rows = [rb0, rb1, rb2]
    sidx = [si0, si1, si2]
    didx = [di0, di1, di2]
    c = lax.axis_index("c")
    s = lax.axis_index("s")
    wid = c * NS + s
    base = wid * EPT

    def fire_si(k, b):
        pltpu.async_copy(src_hbm.at[pl.ds(base + k * CHUNK, CHUNK)],
                         sidx[b], sem_si.at[b])

    def fire_di(k, b):
        pltpu.async_copy(dst_hbm.at[pl.ds(base + k * CHUNK, CHUNK)],
                         didx[b], sem_di.at[b])

    def fire_g(b):
        pltpu.async_copy(feat_hbm.at[sidx[b]], rows[b], sem_g.at[b])

    def wait_si(b):
        pltpu.make_async_copy(src_hbm.at[pl.ds(base, CHUNK)],
                              sidx[b], sem_si.at[b]).wait()

    def wait_di(b):
        pltpu.make_async_copy(dst_hbm.at[pl.ds(base, CHUNK)],
                              didx[b], sem_di.at[b]).wait()

    def wait_g(b):
        pltpu.make_async_copy(feat_hbm.at[sidx[b]],
                              rows[b], sem_g.at[b]).wait()

    def wait_s(b):
        pltpu.make_async_copy(rows[b], accum.at[pl.ds(0, CHUNK)],
                              sem_s.at[b]).wait()

    def step(k, b, *, wait_sprev, do_g, do_si, do_di):
        nb = (b + 2) % NBUF
        wait_g(b)
        if do_si:
            fire_si(k + NBUF, b)
        wait_di(b)
        pltpu.async_copy(rows[b], accum.at[didx[b]], sem_s.at[b], add=True)
        if wait_sprev:
            wait_s(nb)
        if do_di:
            fire_di(k + 2, nb)
        if do_g:
            wait_si(nb)
            fire_g(nb)

    for k in range(NBUF):
        fire_si(k, k)
    for k in range(2):
        fire_di(k, k)
    for k in range(2):
        wait_si(k)
        fire_g(k)

    r0 = s * ROWS_PER_TILE
    pltpu.sync_copy(feat_hbm.at[pl.ds(r0, ROWS_PER_TILE)],
                    accum.at[pl.ds(r0, ROWS_PER_TILE)])

    @pl.when(s == NS - 1)
    def _init_tail():
        t0 = NS * ROWS_PER_TILE
        pltpu.sync_copy(feat_hbm.at[pl.ds(t0, ROWS_TAIL)],
                        accum.at[pl.ds(t0, ROWS_TAIL)])

    plsc.subcore_barrier()

    step(0, 0, wait_sprev=False, do_g=True, do_si=True, do_di=True)
    step(1, 1, wait_sprev=True, do_g=True, do_si=True, do_di=True)
    step(2, 2, wait_sprev=True, do_g=True, do_si=True, do_di=True)

    @pl.loop(NBUF, NFULL - NBUF, step=NBUF)
    def _group(g):
        for b in range(NBUF):
            step(g + b, b, wait_sprev=True, do_g=True, do_si=True,
                 do_di=True)

    step(NFULL - 3, 0, wait_sprev=True, do_g=True, do_si=False, do_di=True)
    step(NFULL - 2, 1, wait_sprev=True, do_g=False, do_si=False, do_di=False)
    step(NFULL - 1, 2, wait_sprev=True, do_g=False, do_si=False, do_di=False)

    t_off = base + NFULL * CHUNK
    pltpu.sync_copy(src_hbm.at[pl.ds(t_off, TAIL)], si0.at[pl.ds(0, TAIL)])
    pltpu.sync_copy(dst_hbm.at[pl.ds(t_off, TAIL)], didx_tail)
    pltpu.async_copy(feat_hbm.at[si0.at[pl.ds(0, TAIL)]],
                     rb0.at[pl.ds(0, TAIL)], sem_g.at[0]).wait()
    pltpu.async_copy(rb0.at[pl.ds(0, TAIL)], accum.at[didx_tail],
                     sem_s.at[0], add=True)

    wait_s(2)
    pltpu.make_async_copy(rb0.at[pl.ds(0, TAIL)], accum.at[pl.ds(0, TAIL)],
                          sem_s.at[0]).wait()

    plsc.subcore_barrier()

    pltpu.sync_copy(accum.at[pl.ds(r0, ROWS_PER_TILE)],
                    out_hbm.at[pl.ds(c * N_NODES + r0, ROWS_PER_TILE)])

    @pl.when(s == NS - 1)
    def _out_tail():
        t0 = NS * ROWS_PER_TILE
        pltpu.sync_copy(accum.at[pl.ds(t0, ROWS_TAIL)],
                        out_hbm.at[pl.ds(c * N_NODES + t0, ROWS_TAIL)])


_COMBINE_BLOCK = 1000
_COMBINE_GRID = N_NODES // _COMBINE_BLOCK


def _combine_body(p0_ref, p1_ref, feat_ref, out_ref):
    out_ref[...] = p0_ref[...] + p1_ref[...] - feat_ref[...]


def _combine(partials, feat):
    return pl.pallas_call(
        _combine_body,
        grid=(_COMBINE_GRID,),
        in_specs=[
            pl.BlockSpec((_COMBINE_BLOCK, D_FEAT), lambda k: (k, 0)),
            pl.BlockSpec((_COMBINE_BLOCK, D_FEAT),
                         lambda k: (k + _COMBINE_GRID, 0)),
            pl.BlockSpec((_COMBINE_BLOCK, D_FEAT), lambda k: (k, 0)),
        ],
        out_specs=pl.BlockSpec((_COMBINE_BLOCK, D_FEAT), lambda k: (k, 0)),
        out_shape=jax.ShapeDtypeStruct((N_NODES, D_FEAT), jnp.float32),
    )(partials, partials, feat)


def kernel(feat, edge_index):
    src = edge_index[0]
    dst = edge_index[1]
    partials = _gin_scatter_sc(feat, src, dst)
    return _combine(partials, feat)
